# speedup vs baseline: 28.4894x; 28.4894x over previous
import functools

import jax
import jax.numpy as jnp
from jax.experimental import pallas as pl

K = 512
_HI_INIT = 0x7F800000


def _topk_mask_block(x_ref, o_ref, *, k):
    x = x_ref[...]
    keys = jax.lax.bitcast_convert_type(x, jnp.int32) & jnp.int32(0x7FFFFFFF)
    rows = x.shape[0]

    def body(_, carry):
        lo, hi = carry
        mid = lo + jax.lax.shift_right_logical(hi - lo, 1)
        cnt = jnp.sum((keys >= mid).astype(jnp.int32), axis=1, keepdims=True)
        ge = cnt >= k
        lo = jnp.where(ge, mid, lo)
        hi = jnp.where(ge, hi, mid)
        return lo, hi

    lo0 = jnp.zeros((rows, 1), jnp.int32)
    hi0 = jnp.full((rows, 1), _HI_INIT, jnp.int32)
    lo, _ = jax.lax.fori_loop(0, 31, body, (lo0, hi0))
    o_ref[...] = jnp.where(keys >= lo, x, jnp.zeros_like(x))


def kernel(input):
    x = input
    B, C, W = x.shape
    rows = B * C
    x2 = x.reshape(rows, W)
    blk = 16
    grid = rows // blk
    out = pl.pallas_call(
        functools.partial(_topk_mask_block, k=K),
        grid=(grid,),
        in_specs=[pl.BlockSpec((blk, W), lambda i: (i, 0))],
        out_specs=pl.BlockSpec((blk, W), lambda i: (i, 0)),
        out_shape=jax.ShapeDtypeStruct((rows, W), x.dtype),
    )(x2)
    return out.reshape(B, C, W)
